# baseline (speedup 1.0000x reference)
import functools

import jax
import jax.numpy as jnp
from jax import lax
from jax.experimental import pallas as pl
from jax.experimental.pallas import tpu as pltpu

H = 256
NGS = 10000
NGC = 10000
E = 160000


def _premix_kernel(gc0_ref, gc1_ref, gs_ref,
                   wss1_ref, bss1_ref, wc1_ref, bc1_ref,
                   wlgc_ref, blgc_ref,
                   pss_s_ref, pss_d_ref, qc_s_ref, qc_d_ref, outfc_ref):
    gs = gs_ref[...]
    gc0 = gc0_ref[...]
    gc1 = gc1_ref[...]
    dot = functools.partial(jnp.dot, preferred_element_type=jnp.float32)
    pss_s_ref[...] = dot(gs, wss1_ref[:H, :])
    pss_d_ref[...] = dot(gs, wss1_ref[H:, :]) + bss1_ref[...]
    qc_s_ref[...] = dot(gc0, wc1_ref[:H, :]) + dot(gc1, wc1_ref[H:2 * H, :])
    qc_d_ref[...] = dot(gs, wc1_ref[2 * H:, :]) + bc1_ref[...]
    outfc_ref[...] = (dot(gc0, wlgc_ref[:H, :]) + dot(gc1, wlgc_ref[H:, :])
                      + blgc_ref[...])


def _premix(nf_gc0, nf_gc1, nf_gs, w_msg_ss1, b_msg_ss1, w_msg_c2s1,
            b_msg_c2s1, w_lin_gc, b_lin_gc):
    n = NGS
    bn = 2000
    grid = (n // bn,)
    full = lambda r, c: pl.BlockSpec((r, c), lambda i: (0, 0))
    rows = lambda c: pl.BlockSpec((bn, c), lambda i: (i, 0))
    return pl.pallas_call(
        _premix_kernel,
        grid=grid,
        in_specs=[rows(H), rows(H), rows(H),
                  full(2 * H, 2 * H), full(1, 2 * H),
                  full(3 * H, 2 * H), full(1, 2 * H),
                  full(2 * H, H), full(1, H)],
        out_specs=[rows(2 * H), rows(2 * H), rows(2 * H), rows(2 * H),
                   rows(H)],
        out_shape=[jax.ShapeDtypeStruct((n, 2 * H), jnp.float32)] * 4
        + [jax.ShapeDtypeStruct((n, H), jnp.float32)],
    )(nf_gc0, nf_gc1, nf_gs, w_msg_ss1, b_msg_ss1.reshape(1, -1),
      w_msg_c2s1, b_msg_c2s1.reshape(1, -1), w_lin_gc, b_lin_gc.reshape(1, -1))


def _edge_mlp_kernel(x_ref, w2f_ref, b2f_ref, w2g_ref, b2g_ref, f_ref):
    h = jnp.maximum(x_ref[...], 0.0)
    dot = functools.partial(jnp.dot, preferred_element_type=jnp.float32)
    m = dot(h, w2f_ref[...]) + b2f_ref[...]
    g = dot(h, w2g_ref[...]) + b2g_ref[...]
    k = jax.nn.sigmoid(g[:, :1])
    f_ref[...] = m * k


def _edge_mlp(x_pre, w_msg2, b_msg2):
    e = x_pre.shape[0]
    be = 1280
    grid = (e // be,)
    w2g = jnp.pad(w_msg2[:, :1], ((0, 0), (0, 127)))
    b2g = jnp.pad(b_msg2[:1], (0, 127)).reshape(1, 128)
    w2f = w_msg2[:, 1:]
    b2f = b_msg2[1:].reshape(1, -1)
    return pl.pallas_call(
        _edge_mlp_kernel,
        grid=grid,
        in_specs=[pl.BlockSpec((be, 2 * H), lambda i: (i, 0)),
                  pl.BlockSpec((2 * H, 4 * H), lambda i: (0, 0)),
                  pl.BlockSpec((1, 4 * H), lambda i: (0, 0)),
                  pl.BlockSpec((2 * H, 128), lambda i: (0, 0)),
                  pl.BlockSpec((1, 128), lambda i: (0, 0))],
        out_specs=pl.BlockSpec((be, 4 * H), lambda i: (i, 0)),
        out_shape=jax.ShapeDtypeStruct((e, 4 * H), jnp.float32),
    )(x_pre, w2f, b2f, w2g, b2g)


def _finish_kernel(gs_ref, s1_ref, m2_ref, m3_ref, s4_ref, dega_ref,
                   t1_ref, u2_ref, u3_ref, t4_ref, degb_ref,
                   wra_ref, bra_ref, wrb_ref, brb_ref, wl_ref, bl_ref,
                   out_ref):
    dot = functools.partial(jnp.dot, preferred_element_type=jnp.float32)
    dega = dega_ref[...]
    degb = degb_ref[...]
    maska = dega > 0.0
    maskb = degb > 0.0
    gs = gs_ref[...]

    def red(x1, x2, x3, x4, mask, deg, w_ref, b_ref):
        n2 = jnp.where(mask, x2, 0.0)
        n3 = jnp.where(mask, x3, 0.0)
        n4 = x4 / jnp.maximum(deg, 1.0)
        return (dot(gs, w_ref[:H, :]) + dot(x1, w_ref[H:2 * H, :])
                + dot(n2, w_ref[2 * H:3 * H, :])
                + dot(n3, w_ref[3 * H:4 * H, :])
                + dot(n4, w_ref[4 * H:, :]) + b_ref[...])

    a = red(s1_ref[...], m2_ref[...], m3_ref[...], s4_ref[...], maska, dega,
            wra_ref, bra_ref)
    b = red(t1_ref[...], u2_ref[...], u3_ref[...], t4_ref[...], maskb, degb,
            wrb_ref, brb_ref)
    out_ref[...] = dot(a + b, wl_ref[...]) + bl_ref[...]


def _finish(x_gs, s1, m2, m3, s4, deg_ss, t1, u2, u3, t4, deg_c2s,
            w_red_ss, b_red_ss, w_red_c2s, b_red_c2s, w_lin_gs, b_lin_gs):
    n = NGS
    bn = 2000
    grid = (n // bn,)
    rows = lambda c: pl.BlockSpec((bn, c), lambda i: (i, 0))
    full = lambda r, c: pl.BlockSpec((r, c), lambda i: (0, 0))
    return pl.pallas_call(
        _finish_kernel,
        grid=grid,
        in_specs=[rows(H), rows(H), rows(H), rows(H), rows(H), rows(1),
                  rows(H), rows(H), rows(H), rows(H), rows(1),
                  full(5 * H, H), full(1, H), full(5 * H, H), full(1, H),
                  full(H, H), full(1, H)],
        out_specs=rows(H),
        out_shape=jax.ShapeDtypeStruct((n, H), jnp.float32),
    )(x_gs, s1, m2, m3, s4, deg_ss.reshape(n, 1),
      t1, u2, u3, t4, deg_c2s.reshape(n, 1),
      w_red_ss, b_red_ss.reshape(1, -1), w_red_c2s, b_red_c2s.reshape(1, -1),
      w_lin_gs, b_lin_gs.reshape(1, -1))


def _segment_all(f, dst):
    f1 = f[:, :H]
    f2 = f[:, H:2 * H]
    f3 = f[:, 2 * H:3 * H]
    f4 = f[:, 3 * H:]
    deg = jax.ops.segment_sum(jnp.ones(dst.shape[0], dtype=jnp.float32), dst,
                              num_segments=NGS)
    s1 = jax.ops.segment_sum(f1, dst, num_segments=NGS)
    m2 = jax.ops.segment_max(f2, dst, num_segments=NGS)
    m3 = jax.ops.segment_min(f3, dst, num_segments=NGS)
    s4 = jax.ops.segment_sum(f4, dst, num_segments=NGS)
    m2 = jnp.where(deg[:, None] > 0, m2, 0.0)
    m3 = jnp.where(deg[:, None] > 0, m3, 0.0)
    return s1, m2, m3, s4, deg


def kernel(nf_gc0, nf_gc1, nf_gs, edge_ss, edge_c2s, w_msg_ss1, b_msg_ss1, w_msg_ss2, b_msg_ss2, w_red_ss, b_red_ss, w_msg_c2s1, b_msg_c2s1, w_msg_c2s2, b_msg_c2s2, w_red_c2s, b_red_c2s, w_lin_gc, b_lin_gc, w_lin_gs, b_lin_gs):
    pss_s, pss_d, qc_s, qc_d, out_fc = _premix(
        nf_gc0, nf_gc1, nf_gs, w_msg_ss1, b_msg_ss1, w_msg_c2s1, b_msg_c2s1,
        w_lin_gc, b_lin_gc)

    s, d = edge_ss[0], edge_ss[1]
    x_pre_ss = pss_s[s] + pss_d[d]
    f_ss = _edge_mlp(x_pre_ss, w_msg_ss2, b_msg_ss2)
    s1, m2, m3, s4, deg_ss = _segment_all(f_ss, d)

    s2, d2 = edge_c2s[0], edge_c2s[1]
    x_pre_c2s = qc_s[s2] + qc_d[d2]
    f_c2s = _edge_mlp(x_pre_c2s, w_msg_c2s2, b_msg_c2s2)
    t1, u2, u3, t4, deg_c2s = _segment_all(f_c2s, d2)

    out_fs = _finish(nf_gs, s1, m2, m3, s4, deg_ss, t1, u2, u3, t4, deg_c2s,
                     w_red_ss, b_red_ss, w_red_c2s, b_red_c2s,
                     w_lin_gs, b_lin_gs)
    return (out_fc, out_fs)


# speedup vs baseline: 1.3558x; 1.3558x over previous
import functools

import jax
import jax.numpy as jnp
from jax import lax
from jax.experimental import pallas as pl
from jax.experimental.pallas import tpu as pltpu
from jax.experimental.pallas import tpu_sc as plsc

H = 256
NGS = 10000
NGC = 10000
E = 160000

NW = 32
RPW = 5
NT = 64
NPAD = NW * RPW * NT
CH = 2000
GF = 32
CAP = CH + 4 * GF
GE = 40

_mesh = functools.partial(plsc.VectorSubcoreMesh,
                          core_axis_name="c", subcore_axis_name="s")


def _premix_kernel(gc0_ref, gc1_ref, gs_ref,
                   wss1_ref, bss1_ref, wc1_ref, bc1_ref,
                   wlgc_ref, blgc_ref,
                   pss_s_ref, pss_d_ref, qc_s_ref, qc_d_ref, outfc_ref):
    gs = gs_ref[...]
    gc0 = gc0_ref[...]
    gc1 = gc1_ref[...]
    dot = functools.partial(jnp.dot, preferred_element_type=jnp.float32)
    pss_s_ref[...] = dot(gs, wss1_ref[:H, :])
    pss_d_ref[...] = dot(gs, wss1_ref[H:, :]) + bss1_ref[...]
    qc_s_ref[...] = dot(gc0, wc1_ref[:H, :]) + dot(gc1, wc1_ref[H:2 * H, :])
    qc_d_ref[...] = dot(gs, wc1_ref[2 * H:, :]) + bc1_ref[...]
    outfc_ref[...] = (dot(gc0, wlgc_ref[:H, :]) + dot(gc1, wlgc_ref[H:, :])
                      + blgc_ref[...])


def _premix(nf_gc0, nf_gc1, nf_gs, w_msg_ss1, b_msg_ss1, w_msg_c2s1,
            b_msg_c2s1, w_lin_gc, b_lin_gc):
    n = NGS
    bn = 2000
    grid = (n // bn,)
    full = lambda r, c: pl.BlockSpec((r, c), lambda i: (0, 0))
    rows = lambda c: pl.BlockSpec((bn, c), lambda i: (i, 0))
    return pl.pallas_call(
        _premix_kernel,
        grid=grid,
        in_specs=[rows(H), rows(H), rows(H),
                  full(2 * H, 2 * H), full(1, 2 * H),
                  full(3 * H, 2 * H), full(1, 2 * H),
                  full(2 * H, H), full(1, H)],
        out_specs=[rows(2 * H), rows(2 * H), rows(2 * H), rows(2 * H),
                   rows(H)],
        out_shape=[jax.ShapeDtypeStruct((n, 2 * H), jnp.float32)] * 4
        + [jax.ShapeDtypeStruct((n, H), jnp.float32)],
    )(nf_gc0, nf_gc1, nf_gs, w_msg_ss1, b_msg_ss1.reshape(1, -1),
      w_msg_c2s1, b_msg_c2s1.reshape(1, -1), w_lin_gc, b_lin_gc.reshape(1, -1))


def _sc_edge_premix(p_s, p_d, esrc, edst):
    pw = E // NW

    @functools.partial(
        pl.kernel,
        mesh=_mesh(num_cores=2),
        out_type=[jax.ShapeDtypeStruct((E, 2 * H), jnp.float32)] * 2,
        scratch_types=[
            pltpu.VMEM((GE,), jnp.int32),
            pltpu.VMEM((GE,), jnp.int32),
            pltpu.VMEM((GE, 2 * H), jnp.float32),
            pltpu.VMEM((GE, 2 * H), jnp.float32),
            pltpu.SemaphoreType.DMA,
            pltpu.SemaphoreType.DMA,
        ],
    )
    def k(ps_hbm, pd_hbm, es_hbm, ed_hbm, outs_hbm, outd_hbm,
          sidx, didx, bufs, bufd, sema, semb):
        wid = lax.axis_index("s") * 2 + lax.axis_index("c")
        base = wid * pw

        def body(b, carry):
            off = base + b * GE
            pltpu.sync_copy(es_hbm.at[pl.ds(off, GE)], sidx)
            pltpu.sync_copy(ed_hbm.at[pl.ds(off, GE)], didx)
            cpa = pltpu.async_copy(ps_hbm.at[sidx], bufs, sema)
            cpb = pltpu.async_copy(pd_hbm.at[didx], bufd, semb)
            cpa.wait()
            pltpu.sync_copy(bufs, outs_hbm.at[pl.ds(off, GE), :])
            cpb.wait()
            pltpu.sync_copy(bufd, outd_hbm.at[pl.ds(off, GE), :])
            return carry

        lax.fori_loop(0, pw // GE, body, 0)

    return k(p_s, p_d, esrc, edst)


BLK = 2000
SLEN = E + 2 * BLK
CAPA = 2 * BLK + 48
CAPB = BLK + 64
TRASH_A = 320 << 20


def _sc_segment_reduce(f, edst):
    @functools.partial(
        pl.kernel,
        mesh=_mesh(num_cores=2),
        out_type=[jax.ShapeDtypeStruct((NPAD, 4 * H), jnp.float32),
                  jax.ShapeDtypeStruct((NPAD, 16), jnp.float32),
                  jax.ShapeDtypeStruct((NW * SLEN,), jnp.int32)],
        scratch_types=[
            pltpu.VMEM((NT + 1, 4 * H), jnp.float32),
            pltpu.VMEM((NT + 1, 16), jnp.float32),
            pltpu.VMEM((BLK,), jnp.int32),
            pltpu.VMEM((CAPA,), jnp.int32),
            pltpu.VMEM((CAPB,), jnp.int32),
            pltpu.VMEM((GF,), jnp.int32),
            pltpu.VMEM((GF, 4 * H), jnp.float32),
            pltpu.SemaphoreType.DMA,
        ],
    )
    def k(f_hbm, ed_hbm, out_hbm, deg_hbm, strip_hbm, acc, dacc, dbuf,
          stage, pkl, gidx, fbuf, sem):
        wid = lax.axis_index("s") * 2 + lax.axis_index("c")
        lo_w = wid * (RPW * NT)
        sbase = wid * SLEN

        def compact(m, pk):
            lane = lax.iota(jnp.int32, 16)
            ps = jnp.where(m, 1, 0)
            for kk in (1, 2, 4, 8):
                sh = ps.at[jnp.maximum(lane - kk, 0)].get(
                    mode="promise_in_bounds")
                ps = ps + jnp.where(lane >= kk, sh, 0)
            cnt = ps[15]
            rank = lane + 1
            idx = jnp.zeros((16,), jnp.int32)
            for s in (8, 4, 2, 1):
                pv = ps.at[idx + (s - 1)].get(mode="promise_in_bounds")
                idx = jnp.where(pv < rank, idx + s, idx)
            cpk = pk.at[idx].get(mode="promise_in_bounds")
            return cpk, cnt

        def scan_chunk(ch, carry):
            off, nblk = carry
            cbase = ch * BLK
            pltpu.sync_copy(ed_hbm.at[pl.ds(cbase, BLK)], dbuf)

            def scan_vec(i, off2):
                lane = lax.iota(jnp.int32, 16)
                trash_a = jnp.full((16,), TRASH_A, jnp.int32)
                dv = dbuf[pl.ds(i * 16, 16)]
                m = (dv >= lo_w) & (dv < lo_w + RPW * NT)
                eidv = cbase + i * 16 + lane
                pk = jnp.where(m, eidv + ((dv - lo_w) << 20), trash_a)
                cpk, cnt = compact(m, pk)
                stage[pl.ds(off2, 16)] = cpk
                return off2 + cnt

            off = lax.fori_loop(0, BLK // 16, scan_vec, off)
            spill = off >= BLK

            @pl.when(spill)
            def _():
                pltpu.sync_copy(stage.at[pl.ds(0, BLK)],
                                strip_hbm.at[pl.ds(sbase + nblk * BLK, BLK)])
                nmv = (off - BLK) // 16 + 2

                def mv(t, c):
                    stage[pl.ds(t * 16, 16)] = stage[pl.ds(BLK + t * 16, 16)]
                    return c

                lax.fori_loop(0, nmv, mv, 0)

            off = jnp.where(spill, off - BLK, off)
            nblk = nblk + jnp.where(spill, 1, 0)
            return (off, nblk)

        off, nblk = lax.fori_loop(0, E // BLK, scan_chunk, (0, 0))
        trash_a = jnp.full((16,), TRASH_A, jnp.int32)
        stage[pl.ds(off, 16)] = trash_a
        stage[pl.ds(off + 16, 16)] = trash_a

        def padrest(t, c):
            stage[pl.ds(t * 16, 16)] = jnp.full((16,), TRASH_A, jnp.int32)
            return c

        lax.fori_loop((off + 32) // 16, BLK // 16, padrest, 0)
        pltpu.sync_copy(stage.at[pl.ds(0, BLK)],
                        strip_hbm.at[pl.ds(sbase + nblk * BLK, BLK)])
        nblk = nblk + 1

        def region(r, carry0):
            lo_r = r * NT

            def initrow(i, c):
                zeros16 = jnp.zeros((16,), jnp.float32)
                neginf = jnp.full((16,), -3.0e38, jnp.float32)
                posinf = jnp.full((16,), 3.0e38, jnp.float32)
                for cc in range(16):
                    acc[i, pl.ds(cc * 16, 16)] = zeros16
                for cc in range(16, 32):
                    acc[i, pl.ds(cc * 16, 16)] = neginf
                for cc in range(32, 48):
                    acc[i, pl.ds(cc * 16, 16)] = posinf
                for cc in range(48, 64):
                    acc[i, pl.ds(cc * 16, 16)] = zeros16
                dacc[i, :] = zeros16
                return c

            lax.fori_loop(0, NT + 1, initrow, 0)

            def flush_batch(kk, c):
                pk0 = pkl[pl.ds(kk * GF, 16)]
                pk1 = pkl[pl.ds(kk * GF + 16, 16)]
                gidx[pl.ds(0, 16)] = pk0 & 0xFFFFF
                gidx[pl.ds(16, 16)] = pk1 & 0xFFFFF
                pltpu.async_copy(f_hbm.at[gidx], fbuf, sem).wait()
                for half in (0, 1):

                    def acc_one(j2, c2, half=half):
                        lane = lax.iota(jnp.int32, 16)
                        one0 = jnp.where(lane == 0, 1.0, 0.0)
                        pkv_vec = pkl[pl.ds(kk * GF + half * 16 + j2, 16)]
                        dloc = (pkv_vec[0] >> 20) - lo_r
                        j = half * 16 + j2
                        for cc in range(16):
                            sl = pl.ds(cc * 16, 16)
                            acc[dloc, sl] = acc[dloc, sl] + fbuf[j, sl]
                        for cc in range(16, 32):
                            sl = pl.ds(cc * 16, 16)
                            acc[dloc, sl] = jnp.maximum(acc[dloc, sl],
                                                        fbuf[j, sl])
                        for cc in range(32, 48):
                            sl = pl.ds(cc * 16, 16)
                            acc[dloc, sl] = jnp.minimum(acc[dloc, sl],
                                                        fbuf[j, sl])
                        for cc in range(48, 64):
                            sl = pl.ds(cc * 16, 16)
                            acc[dloc, sl] = acc[dloc, sl] + fbuf[j, sl]
                        dacc[dloc, :] = dacc[dloc, :] + one0
                        return c2

                    lax.fori_loop(0, 16, acc_one, 0)
                return c

            def block_body(b, offb):
                pltpu.sync_copy(
                    strip_hbm.at[pl.ds(sbase + b * BLK, BLK)], dbuf)

                def filt_vec(i, off2):
                    pk = dbuf[pl.ds(i * 16, 16)]
                    dl9 = pk >> 20
                    m2 = (dl9 >= lo_r) & (dl9 < lo_r + NT)
                    cpk, cnt = compact(m2, pk)
                    pkl[pl.ds(off2, 16)] = cpk
                    return off2 + cnt

                offb = lax.fori_loop(0, BLK // 16, filt_vec, offb)
                nfull = offb // GF
                lax.fori_loop(0, nfull, flush_batch, 0)

                @pl.when(nfull > 0)
                def _():
                    for t in range(GF // 16):
                        v_t = pkl[pl.ds(nfull * GF + t * 16, 16)]
                        pkl[pl.ds(t * 16, 16)] = v_t

                return offb - nfull * GF

            offb = lax.fori_loop(0, nblk, block_body, 0)
            pad_r = jnp.full((16,), (lo_r + NT) << 20, jnp.int32)
            pkl[pl.ds(offb, 16)] = pad_r
            pkl[pl.ds(offb + 16, 16)] = pad_r
            nfinal = (offb + GF - 1) // GF
            lax.fori_loop(0, nfinal, flush_batch, 0)

            v = wid * RPW + r
            pltpu.sync_copy(acc.at[pl.ds(0, NT), :],
                            out_hbm.at[pl.ds(v * NT, NT), :])
            pltpu.sync_copy(dacc.at[pl.ds(0, NT), :],
                            deg_hbm.at[pl.ds(v * NT, NT), :])
            return carry0

        lax.fori_loop(0, RPW, region, 0)

    return k(f, edst)


def _edge_mlp_kernel(xs_ref, xd_ref, w2f_ref, b2f_ref, w2g_ref, b2g_ref,
                     f_ref):
    h = jnp.maximum(xs_ref[...] + xd_ref[...], 0.0)
    dot = functools.partial(jnp.dot, preferred_element_type=jnp.float32)
    m = dot(h, w2f_ref[...]) + b2f_ref[...]
    g = dot(h, w2g_ref[...]) + b2g_ref[...]
    kgate = jax.nn.sigmoid(g[:, :1])
    f_ref[...] = m * kgate


def _edge_mlp(x_pre_s, x_pre_d, w_msg2, b_msg2):
    e = x_pre_s.shape[0]
    be = 1280
    grid = (e // be,)
    w2g = jnp.pad(w_msg2[:, :1], ((0, 0), (0, 127)))
    b2g = jnp.pad(b_msg2[:1], (0, 127)).reshape(1, 128)
    w2f = w_msg2[:, 1:]
    b2f = b_msg2[1:].reshape(1, -1)
    return pl.pallas_call(
        _edge_mlp_kernel,
        grid=grid,
        in_specs=[pl.BlockSpec((be, 2 * H), lambda i: (i, 0)),
                  pl.BlockSpec((be, 2 * H), lambda i: (i, 0)),
                  pl.BlockSpec((2 * H, 4 * H), lambda i: (0, 0)),
                  pl.BlockSpec((1, 4 * H), lambda i: (0, 0)),
                  pl.BlockSpec((2 * H, 128), lambda i: (0, 0)),
                  pl.BlockSpec((1, 128), lambda i: (0, 0))],
        out_specs=pl.BlockSpec((be, 4 * H), lambda i: (i, 0)),
        out_shape=jax.ShapeDtypeStruct((e, 4 * H), jnp.float32),
    )(x_pre_s, x_pre_d, w2f, b2f, w2g, b2g)


def _finish_kernel(gs_ref, pa_s1, pa_m2, pa_m3, pa_s4, dega_ref,
                   pb_s1, pb_m2, pb_m3, pb_s4, degb_ref,
                   wra_ref, bra_ref, wrb_ref, brb_ref, wl_ref, bl_ref,
                   out_ref):
    dot = functools.partial(jnp.dot, preferred_element_type=jnp.float32)
    dega = dega_ref[:, :1]
    degb = degb_ref[:, :1]
    gs = gs_ref[...]

    def red(x1, x2, x3, x4, deg, w_ref, b_ref):
        mask = deg > 0.0
        n2 = jnp.where(mask, x2, 0.0)
        n3 = jnp.where(mask, x3, 0.0)
        n4 = x4 / jnp.maximum(deg, 1.0)
        return (dot(gs, w_ref[:H, :]) + dot(x1, w_ref[H:2 * H, :])
                + dot(n2, w_ref[2 * H:3 * H, :])
                + dot(n3, w_ref[3 * H:4 * H, :])
                + dot(n4, w_ref[4 * H:, :]) + b_ref[...])

    a = red(pa_s1[...], pa_m2[...], pa_m3[...], pa_s4[...], dega,
            wra_ref, bra_ref)
    b = red(pb_s1[...], pb_m2[...], pb_m3[...], pb_s4[...], degb,
            wrb_ref, brb_ref)
    out_ref[...] = dot(a + b, wl_ref[...]) + bl_ref[...]


def _finish(x_gs, packed_a, deg_a, packed_b, deg_b,
            w_red_ss, b_red_ss, w_red_c2s, b_red_c2s, w_lin_gs, b_lin_gs):
    n = NGS
    bn = 2000
    grid = (n // bn,)
    rows = lambda c: pl.BlockSpec((bn, c), lambda i: (i, 0))
    colb = lambda j: pl.BlockSpec((bn, H), lambda i, j=j: (i, j))
    full = lambda r, c: pl.BlockSpec((r, c), lambda i: (0, 0))
    return pl.pallas_call(
        _finish_kernel,
        grid=grid,
        in_specs=[rows(H), colb(0), colb(1), colb(2), colb(3), rows(16),
                  colb(0), colb(1), colb(2), colb(3), rows(16),
                  full(5 * H, H), full(1, H), full(5 * H, H), full(1, H),
                  full(H, H), full(1, H)],
        out_specs=rows(H),
        out_shape=jax.ShapeDtypeStruct((n, H), jnp.float32),
    )(x_gs, packed_a, packed_a, packed_a, packed_a, deg_a,
      packed_b, packed_b, packed_b, packed_b, deg_b,
      w_red_ss, b_red_ss.reshape(1, -1), w_red_c2s, b_red_c2s.reshape(1, -1),
      w_lin_gs, b_lin_gs.reshape(1, -1))


def kernel(nf_gc0, nf_gc1, nf_gs, edge_ss, edge_c2s, w_msg_ss1, b_msg_ss1, w_msg_ss2, b_msg_ss2, w_red_ss, b_red_ss, w_msg_c2s1, b_msg_c2s1, w_msg_c2s2, b_msg_c2s2, w_red_c2s, b_red_c2s, w_lin_gc, b_lin_gc, w_lin_gs, b_lin_gs):
    pss_s, pss_d, qc_s, qc_d, out_fc = _premix(
        nf_gc0, nf_gc1, nf_gs, w_msg_ss1, b_msg_ss1, w_msg_c2s1, b_msg_c2s1,
        w_lin_gc, b_lin_gc)

    src_ss, dst_ss = edge_ss[0], edge_ss[1]
    src_c2s, dst_c2s = edge_c2s[0], edge_c2s[1]

    gs_ss, gd_ss = _sc_edge_premix(pss_s, pss_d, src_ss, dst_ss)
    f_ss = _edge_mlp(gs_ss, gd_ss, w_msg_ss2, b_msg_ss2)
    packed_a, deg_a, _sa = _sc_segment_reduce(f_ss, dst_ss)

    gs_c2s, gd_c2s = _sc_edge_premix(qc_s, qc_d, src_c2s, dst_c2s)
    f_c2s = _edge_mlp(gs_c2s, gd_c2s, w_msg_c2s2, b_msg_c2s2)
    packed_b, deg_b, _sb = _sc_segment_reduce(f_c2s, dst_c2s)

    out_fs = _finish(nf_gs, packed_a, deg_a, packed_b, deg_b,
                     w_red_ss, b_red_ss, w_red_c2s, b_red_c2s,
                     w_lin_gs, b_lin_gs)
    return (out_fc, out_fs)


# speedup vs baseline: 1.5125x; 1.1155x over previous
import functools

import jax
import jax.numpy as jnp
from jax import lax
from jax.experimental import pallas as pl
from jax.experimental.pallas import tpu as pltpu
from jax.experimental.pallas import tpu_sc as plsc

H = 256
NGS = 10000
NGC = 10000
E = 160000

NW = 32
RPW = 5
NT = 64
NPAD = NW * RPW * NT
CH = 2000
GF = 32
CAP = CH + 4 * GF
GE = 40

_mesh = functools.partial(plsc.VectorSubcoreMesh,
                          core_axis_name="c", subcore_axis_name="s")


def _premix_kernel(gc0_ref, gc1_ref, gs_ref,
                   wss1_ref, bss1_ref, wc1_ref, bc1_ref,
                   wlgc_ref, blgc_ref,
                   pss_s_ref, pss_d_ref, qc_s_ref, qc_d_ref, outfc_ref):
    gs = gs_ref[...]
    gc0 = gc0_ref[...]
    gc1 = gc1_ref[...]
    dot = functools.partial(jnp.dot, preferred_element_type=jnp.float32)
    pss_s_ref[...] = dot(gs, wss1_ref[:H, :])
    pss_d_ref[...] = dot(gs, wss1_ref[H:, :]) + bss1_ref[...]
    qc_s_ref[...] = dot(gc0, wc1_ref[:H, :]) + dot(gc1, wc1_ref[H:2 * H, :])
    qc_d_ref[...] = dot(gs, wc1_ref[2 * H:, :]) + bc1_ref[...]
    outfc_ref[...] = (dot(gc0, wlgc_ref[:H, :]) + dot(gc1, wlgc_ref[H:, :])
                      + blgc_ref[...])


def _premix(nf_gc0, nf_gc1, nf_gs, w_msg_ss1, b_msg_ss1, w_msg_c2s1,
            b_msg_c2s1, w_lin_gc, b_lin_gc):
    n = NGS
    bn = 2000
    grid = (n // bn,)
    full = lambda r, c: pl.BlockSpec((r, c), lambda i: (0, 0))
    rows = lambda c: pl.BlockSpec((bn, c), lambda i: (i, 0))
    return pl.pallas_call(
        _premix_kernel,
        grid=grid,
        in_specs=[rows(H), rows(H), rows(H),
                  full(2 * H, 2 * H), full(1, 2 * H),
                  full(3 * H, 2 * H), full(1, 2 * H),
                  full(2 * H, H), full(1, H)],
        out_specs=[rows(2 * H), rows(2 * H), rows(2 * H), rows(2 * H),
                   rows(H)],
        out_shape=[jax.ShapeDtypeStruct((n, 2 * H), jnp.float32)] * 4
        + [jax.ShapeDtypeStruct((n, H), jnp.float32)],
    )(nf_gc0, nf_gc1, nf_gs, w_msg_ss1, b_msg_ss1.reshape(1, -1),
      w_msg_c2s1, b_msg_c2s1.reshape(1, -1), w_lin_gc, b_lin_gc.reshape(1, -1))


def _sc_edge_premix(p_s, p_d, esrc, edst):
    pw = E // NW

    @functools.partial(
        pl.kernel,
        mesh=_mesh(num_cores=2),
        out_type=[jax.ShapeDtypeStruct((E, 2 * H), jnp.float32)] * 2,
        scratch_types=[
            pltpu.VMEM((GE,), jnp.int32),
            pltpu.VMEM((GE,), jnp.int32),
            pltpu.VMEM((GE,), jnp.int32),
            pltpu.VMEM((GE,), jnp.int32),
            pltpu.VMEM((GE, 2 * H), jnp.float32),
            pltpu.VMEM((GE, 2 * H), jnp.float32),
            pltpu.VMEM((GE, 2 * H), jnp.float32),
            pltpu.VMEM((GE, 2 * H), jnp.float32),
            pltpu.SemaphoreType.DMA,
            pltpu.SemaphoreType.DMA,
            pltpu.SemaphoreType.DMA,
            pltpu.SemaphoreType.DMA,
        ],
    )
    def k(ps_hbm, pd_hbm, es_hbm, ed_hbm, outs_hbm, outd_hbm,
          sidx0, sidx1, didx0, didx1, bufs0, bufs1, bufd0, bufd1,
          s0, s1, s2, s3):
        wid = lax.axis_index("s") * 2 + lax.axis_index("c")
        base = wid * pw

        def body(i, carry):
            off0 = base + (2 * i) * GE
            off1 = off0 + GE
            ci0 = pltpu.async_copy(es_hbm.at[pl.ds(off0, GE)], sidx0, s0)
            cj0 = pltpu.async_copy(ed_hbm.at[pl.ds(off0, GE)], didx0, s2)
            ci1 = pltpu.async_copy(es_hbm.at[pl.ds(off1, GE)], sidx1, s1)
            cj1 = pltpu.async_copy(ed_hbm.at[pl.ds(off1, GE)], didx1, s3)
            ci0.wait()
            cj0.wait()
            g0s = pltpu.async_copy(ps_hbm.at[sidx0], bufs0, s0)
            g0d = pltpu.async_copy(pd_hbm.at[didx0], bufd0, s2)
            ci1.wait()
            cj1.wait()
            g1s = pltpu.async_copy(ps_hbm.at[sidx1], bufs1, s1)
            g1d = pltpu.async_copy(pd_hbm.at[didx1], bufd1, s3)
            g0s.wait()
            g0d.wait()
            o0s = pltpu.async_copy(bufs0, outs_hbm.at[pl.ds(off0, GE), :], s0)
            o0d = pltpu.async_copy(bufd0, outd_hbm.at[pl.ds(off0, GE), :], s2)
            g1s.wait()
            g1d.wait()
            o1s = pltpu.async_copy(bufs1, outs_hbm.at[pl.ds(off1, GE), :], s1)
            o1d = pltpu.async_copy(bufd1, outd_hbm.at[pl.ds(off1, GE), :], s3)
            o0s.wait()
            o0d.wait()
            o1s.wait()
            o1d.wait()
            return carry

        nb = pw // GE
        lax.fori_loop(0, nb // 2, body, 0)

        offz = base + (nb - 1) * GE
        ciz = pltpu.async_copy(es_hbm.at[pl.ds(offz, GE)], sidx0, s0)
        cjz = pltpu.async_copy(ed_hbm.at[pl.ds(offz, GE)], didx0, s2)
        ciz.wait()
        cjz.wait()
        gzs = pltpu.async_copy(ps_hbm.at[sidx0], bufs0, s0)
        gzd = pltpu.async_copy(pd_hbm.at[didx0], bufd0, s2)
        gzs.wait()
        gzd.wait()
        ozs = pltpu.async_copy(bufs0, outs_hbm.at[pl.ds(offz, GE), :], s0)
        ozd = pltpu.async_copy(bufd0, outd_hbm.at[pl.ds(offz, GE), :], s2)
        ozs.wait()
        ozd.wait()

    return k(p_s, p_d, esrc, edst)


BLK = 2000
SLEN = E + 2 * BLK
CAPA = 2 * BLK + 48
CAPB = BLK + 64
TRASH_A = 320 << 20


def _sc_segment_reduce(f, edst):
    @functools.partial(
        pl.kernel,
        mesh=_mesh(num_cores=2),
        out_type=[jax.ShapeDtypeStruct((NPAD, 4 * H), jnp.float32),
                  jax.ShapeDtypeStruct((NPAD, 16), jnp.float32),
                  jax.ShapeDtypeStruct((NW * SLEN,), jnp.int32)],
        scratch_types=[
            pltpu.VMEM((NT + 1, 4 * H), jnp.float32),
            pltpu.VMEM((NT + 1, 16), jnp.float32),
            pltpu.VMEM((BLK,), jnp.int32),
            pltpu.VMEM((CAPA,), jnp.int32),
            pltpu.VMEM((CAPB,), jnp.int32),
            pltpu.VMEM((GF,), jnp.int32),
            pltpu.VMEM((GF, 4 * H), jnp.float32),
            pltpu.SemaphoreType.DMA,
        ],
    )
    def k(f_hbm, ed_hbm, out_hbm, deg_hbm, strip_hbm, acc, dacc, dbuf,
          stage, pkl, gidx, fbuf, sem):
        wid = lax.axis_index("s") * 2 + lax.axis_index("c")
        lo_w = wid * (RPW * NT)
        sbase = wid * SLEN

        def compact(m, pk):
            lane = lax.iota(jnp.int32, 16)
            ps = jnp.where(m, 1, 0)
            for kk in (1, 2, 4, 8):
                sh = ps.at[jnp.maximum(lane - kk, 0)].get(
                    mode="promise_in_bounds")
                ps = ps + jnp.where(lane >= kk, sh, 0)
            cnt = ps[15]
            rank = lane + 1
            idx = jnp.zeros((16,), jnp.int32)
            for s in (8, 4, 2, 1):
                pv = ps.at[idx + (s - 1)].get(mode="promise_in_bounds")
                idx = jnp.where(pv < rank, idx + s, idx)
            cpk = pk.at[idx].get(mode="promise_in_bounds")
            return cpk, cnt

        def scan_chunk(ch, carry):
            off, nblk = carry
            cbase = ch * BLK
            pltpu.sync_copy(ed_hbm.at[pl.ds(cbase, BLK)], dbuf)

            def scan_vec(i, off2):
                lane = lax.iota(jnp.int32, 16)
                trash_a = jnp.full((16,), TRASH_A, jnp.int32)
                dv = dbuf[pl.ds(i * 16, 16)]
                m = (dv >= lo_w) & (dv < lo_w + RPW * NT)
                eidv = cbase + i * 16 + lane
                pk = jnp.where(m, eidv + ((dv - lo_w) << 20), trash_a)
                cpk, cnt = compact(m, pk)
                stage[pl.ds(off2, 16)] = cpk
                return off2 + cnt

            off = lax.fori_loop(0, BLK // 16, scan_vec, off)
            spill = off >= BLK

            @pl.when(spill)
            def _():
                pltpu.sync_copy(stage.at[pl.ds(0, BLK)],
                                strip_hbm.at[pl.ds(sbase + nblk * BLK, BLK)])
                nmv = (off - BLK) // 16 + 2

                def mv(t, c):
                    stage[pl.ds(t * 16, 16)] = stage[pl.ds(BLK + t * 16, 16)]
                    return c

                lax.fori_loop(0, nmv, mv, 0)

            off = jnp.where(spill, off - BLK, off)
            nblk = nblk + jnp.where(spill, 1, 0)
            return (off, nblk)

        off, nblk = lax.fori_loop(0, E // BLK, scan_chunk, (0, 0))
        trash_a = jnp.full((16,), TRASH_A, jnp.int32)
        stage[pl.ds(off, 16)] = trash_a
        stage[pl.ds(off + 16, 16)] = trash_a

        def padrest(t, c):
            stage[pl.ds(t * 16, 16)] = jnp.full((16,), TRASH_A, jnp.int32)
            return c

        lax.fori_loop((off + 32) // 16, BLK // 16, padrest, 0)
        pltpu.sync_copy(stage.at[pl.ds(0, BLK)],
                        strip_hbm.at[pl.ds(sbase + nblk * BLK, BLK)])
        nblk = nblk + 1

        def region(r, carry0):
            lo_r = r * NT

            def initrow(i, c):
                zeros16 = jnp.zeros((16,), jnp.float32)
                neginf = jnp.full((16,), -3.0e38, jnp.float32)
                posinf = jnp.full((16,), 3.0e38, jnp.float32)
                for cc in range(16):
                    acc[i, pl.ds(cc * 16, 16)] = zeros16
                for cc in range(16, 32):
                    acc[i, pl.ds(cc * 16, 16)] = neginf
                for cc in range(32, 48):
                    acc[i, pl.ds(cc * 16, 16)] = posinf
                for cc in range(48, 64):
                    acc[i, pl.ds(cc * 16, 16)] = zeros16
                dacc[i, :] = zeros16
                return c

            lax.fori_loop(0, NT + 1, initrow, 0)

            def flush_batch(kk, c):
                pk0 = pkl[pl.ds(kk * GF, 16)]
                pk1 = pkl[pl.ds(kk * GF + 16, 16)]
                gidx[pl.ds(0, 16)] = pk0 & 0xFFFFF
                gidx[pl.ds(16, 16)] = pk1 & 0xFFFFF
                pltpu.async_copy(f_hbm.at[gidx], fbuf, sem).wait()
                for half in (0, 1):

                    def acc_one(j2, c2, half=half):
                        lane = lax.iota(jnp.int32, 16)
                        one0 = jnp.where(lane == 0, 1.0, 0.0)
                        pkv_vec = pkl[pl.ds(kk * GF + half * 16 + j2, 16)]
                        dloc = (pkv_vec[0] >> 20) - lo_r
                        j = half * 16 + j2
                        for cc in range(16):
                            sl = pl.ds(cc * 16, 16)
                            plsc.addupdate(acc.at[dloc, sl], fbuf[j, sl])
                        for cc in range(16, 32):
                            sl = pl.ds(cc * 16, 16)
                            acc[dloc, sl] = jnp.maximum(acc[dloc, sl],
                                                        fbuf[j, sl])
                        for cc in range(32, 48):
                            sl = pl.ds(cc * 16, 16)
                            acc[dloc, sl] = jnp.minimum(acc[dloc, sl],
                                                        fbuf[j, sl])
                        for cc in range(48, 64):
                            sl = pl.ds(cc * 16, 16)
                            plsc.addupdate(acc.at[dloc, sl], fbuf[j, sl])
                        plsc.addupdate(dacc.at[dloc, :], one0)
                        return c2

                    lax.fori_loop(0, 16, acc_one, 0)
                return c

            def block_body(b, offb):
                pltpu.sync_copy(
                    strip_hbm.at[pl.ds(sbase + b * BLK, BLK)], dbuf)

                def filt_vec(i, off2):
                    pk = dbuf[pl.ds(i * 16, 16)]
                    dl9 = pk >> 20
                    m2 = (dl9 >= lo_r) & (dl9 < lo_r + NT)
                    cpk, cnt = compact(m2, pk)
                    pkl[pl.ds(off2, 16)] = cpk
                    return off2 + cnt

                offb = lax.fori_loop(0, BLK // 16, filt_vec, offb)
                nfull = offb // GF
                lax.fori_loop(0, nfull, flush_batch, 0)

                @pl.when(nfull > 0)
                def _():
                    for t in range(GF // 16):
                        v_t = pkl[pl.ds(nfull * GF + t * 16, 16)]
                        pkl[pl.ds(t * 16, 16)] = v_t

                return offb - nfull * GF

            offb = lax.fori_loop(0, nblk, block_body, 0)
            pad_r = jnp.full((16,), (lo_r + NT) << 20, jnp.int32)
            pkl[pl.ds(offb, 16)] = pad_r
            pkl[pl.ds(offb + 16, 16)] = pad_r
            nfinal = (offb + GF - 1) // GF
            lax.fori_loop(0, nfinal, flush_batch, 0)

            v = wid * RPW + r
            pltpu.sync_copy(acc.at[pl.ds(0, NT), :],
                            out_hbm.at[pl.ds(v * NT, NT), :])
            pltpu.sync_copy(dacc.at[pl.ds(0, NT), :],
                            deg_hbm.at[pl.ds(v * NT, NT), :])
            return carry0

        lax.fori_loop(0, RPW, region, 0)

    return k(f, edst)


def _edge_mlp_kernel(xs_ref, xd_ref, w2f_ref, b2f_ref, w2g_ref, b2g_ref,
                     f_ref):
    h = jnp.maximum(xs_ref[...] + xd_ref[...], 0.0)
    dot = functools.partial(jnp.dot, preferred_element_type=jnp.float32)
    m = dot(h, w2f_ref[...]) + b2f_ref[...]
    g = dot(h, w2g_ref[...]) + b2g_ref[...]
    kgate = jax.nn.sigmoid(g[:, :1])
    f_ref[...] = m * kgate


def _edge_mlp(x_pre_s, x_pre_d, w_msg2, b_msg2):
    e = x_pre_s.shape[0]
    be = 1280
    grid = (e // be,)
    w2g = jnp.pad(w_msg2[:, :1], ((0, 0), (0, 127)))
    b2g = jnp.pad(b_msg2[:1], (0, 127)).reshape(1, 128)
    w2f = w_msg2[:, 1:]
    b2f = b_msg2[1:].reshape(1, -1)
    return pl.pallas_call(
        _edge_mlp_kernel,
        grid=grid,
        in_specs=[pl.BlockSpec((be, 2 * H), lambda i: (i, 0)),
                  pl.BlockSpec((be, 2 * H), lambda i: (i, 0)),
                  pl.BlockSpec((2 * H, 4 * H), lambda i: (0, 0)),
                  pl.BlockSpec((1, 4 * H), lambda i: (0, 0)),
                  pl.BlockSpec((2 * H, 128), lambda i: (0, 0)),
                  pl.BlockSpec((1, 128), lambda i: (0, 0))],
        out_specs=pl.BlockSpec((be, 4 * H), lambda i: (i, 0)),
        out_shape=jax.ShapeDtypeStruct((e, 4 * H), jnp.float32),
    )(x_pre_s, x_pre_d, w2f, b2f, w2g, b2g)


def _finish_kernel(gs_ref, pa_s1, pa_m2, pa_m3, pa_s4, dega_ref,
                   pb_s1, pb_m2, pb_m3, pb_s4, degb_ref,
                   wra_ref, bra_ref, wrb_ref, brb_ref, wl_ref, bl_ref,
                   out_ref):
    dot = functools.partial(jnp.dot, preferred_element_type=jnp.float32)
    dega = dega_ref[:, :1]
    degb = degb_ref[:, :1]
    gs = gs_ref[...]

    def red(x1, x2, x3, x4, deg, w_ref, b_ref):
        mask = deg > 0.0
        n2 = jnp.where(mask, x2, 0.0)
        n3 = jnp.where(mask, x3, 0.0)
        n4 = x4 / jnp.maximum(deg, 1.0)
        return (dot(gs, w_ref[:H, :]) + dot(x1, w_ref[H:2 * H, :])
                + dot(n2, w_ref[2 * H:3 * H, :])
                + dot(n3, w_ref[3 * H:4 * H, :])
                + dot(n4, w_ref[4 * H:, :]) + b_ref[...])

    a = red(pa_s1[...], pa_m2[...], pa_m3[...], pa_s4[...], dega,
            wra_ref, bra_ref)
    b = red(pb_s1[...], pb_m2[...], pb_m3[...], pb_s4[...], degb,
            wrb_ref, brb_ref)
    out_ref[...] = dot(a + b, wl_ref[...]) + bl_ref[...]


def _finish(x_gs, packed_a, deg_a, packed_b, deg_b,
            w_red_ss, b_red_ss, w_red_c2s, b_red_c2s, w_lin_gs, b_lin_gs):
    n = NGS
    bn = 2000
    grid = (n // bn,)
    rows = lambda c: pl.BlockSpec((bn, c), lambda i: (i, 0))
    colb = lambda j: pl.BlockSpec((bn, H), lambda i, j=j: (i, j))
    full = lambda r, c: pl.BlockSpec((r, c), lambda i: (0, 0))
    return pl.pallas_call(
        _finish_kernel,
        grid=grid,
        in_specs=[rows(H), colb(0), colb(1), colb(2), colb(3), rows(16),
                  colb(0), colb(1), colb(2), colb(3), rows(16),
                  full(5 * H, H), full(1, H), full(5 * H, H), full(1, H),
                  full(H, H), full(1, H)],
        out_specs=rows(H),
        out_shape=jax.ShapeDtypeStruct((n, H), jnp.float32),
    )(x_gs, packed_a, packed_a, packed_a, packed_a, deg_a,
      packed_b, packed_b, packed_b, packed_b, deg_b,
      w_red_ss, b_red_ss.reshape(1, -1), w_red_c2s, b_red_c2s.reshape(1, -1),
      w_lin_gs, b_lin_gs.reshape(1, -1))


def kernel(nf_gc0, nf_gc1, nf_gs, edge_ss, edge_c2s, w_msg_ss1, b_msg_ss1, w_msg_ss2, b_msg_ss2, w_red_ss, b_red_ss, w_msg_c2s1, b_msg_c2s1, w_msg_c2s2, b_msg_c2s2, w_red_c2s, b_red_c2s, w_lin_gc, b_lin_gc, w_lin_gs, b_lin_gs):
    pss_s, pss_d, qc_s, qc_d, out_fc = _premix(
        nf_gc0, nf_gc1, nf_gs, w_msg_ss1, b_msg_ss1, w_msg_c2s1, b_msg_c2s1,
        w_lin_gc, b_lin_gc)

    src_ss, dst_ss = edge_ss[0], edge_ss[1]
    src_c2s, dst_c2s = edge_c2s[0], edge_c2s[1]

    gs_ss, gd_ss = _sc_edge_premix(pss_s, pss_d, src_ss, dst_ss)
    f_ss = _edge_mlp(gs_ss, gd_ss, w_msg_ss2, b_msg_ss2)
    packed_a, deg_a, _sa = _sc_segment_reduce(f_ss, dst_ss)

    gs_c2s, gd_c2s = _sc_edge_premix(qc_s, qc_d, src_c2s, dst_c2s)
    f_c2s = _edge_mlp(gs_c2s, gd_c2s, w_msg_c2s2, b_msg_c2s2)
    packed_b, deg_b, _sb = _sc_segment_reduce(f_c2s, dst_c2s)

    out_fs = _finish(nf_gs, packed_a, deg_a, packed_b, deg_b,
                     w_red_ss, b_red_ss, w_red_c2s, b_red_c2s,
                     w_lin_gs, b_lin_gs)
    return (out_fc, out_fs)
